# Initial kernel scaffold; baseline (speedup 1.0000x reference)
import jax
import jax.numpy as jnp
from jax.experimental import pallas as pl


def kernel(inputs, W, digit_caps):
    raise NotImplementedError("write your pallas kernel here")


# speedup vs baseline: 2.1678x; 2.1678x over previous
import functools

import jax
import jax.numpy as jnp
from jax import lax
from jax.experimental import pallas as pl
from jax.experimental.pallas import tpu as pltpu
from jax.experimental.pallas import tpu_sc as plsc

_B, _N, _ID = 16, 8192, 8
_C, _D, _M = 10, 8, 4
_K = _D * _M
_NM = _N * _M
_JB = 512
_NW = 32


def _tc_body(xt_ref, wt_ref, ut_ref, usum_ref):
    xb = xt_ref[...].astype(jnp.bfloat16).astype(jnp.float32)
    wb = wt_ref[...].astype(jnp.bfloat16).astype(jnp.float32)
    acc = wb[0][:, None, :] * xb[0][None, :, :]
    for i in range(1, _ID):
        acc = acc + wb[i][:, None, :] * xb[i][None, :, :]
    ut_ref[...] = acc

    @pl.when(pl.program_id(0) == 0)
    def _():
        usum_ref[...] = jnp.zeros_like(usum_ref)

    usum_ref[...] += jnp.sum(acc, axis=2)


_tc_call = pl.pallas_call(
    _tc_body,
    grid=(_N // _JB,),
    in_specs=[
        pl.BlockSpec((_ID, _B, _JB), lambda j: (0, 0, j)),
        pl.BlockSpec((_ID, _K, _JB), lambda j: (0, 0, j)),
    ],
    out_specs=[
        pl.BlockSpec((_K, _B, _JB), lambda j: (0, 0, j)),
        pl.BlockSpec((_K, _B), lambda j: (0, 0)),
    ],
    out_shape=[
        jax.ShapeDtypeStruct((_K, _B, _N), jnp.float32),
        jax.ShapeDtypeStruct((_K, _B), jnp.float32),
    ],
)


_U = 4


def _sc_body(ut_hbm, dc_hbm, psum_hbm, pcnt_hbm, buf, acc, cntacc, dcv, dcsp):
    wid = lax.axis_index("s") * 2 + lax.axis_index("c")
    lanes = jnp.arange(16, dtype=jnp.int32)
    ones = jnp.ones((16,), jnp.float32)

    pltpu.sync_copy(dc_hbm, dcv.at[pl.ds(16, _C * _D)])
    for t in range(_C * _D):
        splat = plsc.load_gather(dcv, [jnp.full((16,), 16 + t, jnp.int32)])
        dcsp[t, :] = splat

    for t in range(_D * _C):
        acc[pl.ds(t * 16, 16)] = jnp.zeros((16,), jnp.float32)
    for c in range(_C):
        cntacc[pl.ds(c * 16, 16)] = jnp.zeros((16,), jnp.float32)

    for cc in range(2):
        combo = wid * 2 + cc
        b = combo // _M
        m = combo % _M
        pltpu.sync_copy(ut_hbm.at[pl.ds(m * _D, _D), pl.ds(b, 1), :], buf)

        def gbody(g, carry):
            base = g * (16 * _U)
            vd = [[buf[dd, 0, pl.ds(base + u * 16, 16)] for dd in range(_D)]
                  for u in range(_U)]
            best = [None] * _U
            widx = [jnp.zeros((16,), jnp.int32) for _ in range(_U)]
            for c in range(_C):
                s = [None] * _U
                for dd in range(_D):
                    coeff = dcsp[c * _D + dd, :]
                    for u in range(_U):
                        t = vd[u][dd] * coeff
                        s[u] = t if dd == 0 else s[u] + t
                if c == 0:
                    best = s
                else:
                    cvec = jnp.full((16,), c, jnp.int32)
                    for u in range(_U):
                        gt = s[u] > best[u]
                        best[u] = jnp.where(gt, s[u], best[u])
                        widx[u] = jnp.where(gt, cvec, widx[u])
            for u in range(_U):
                widx16 = widx[u] * 16 + lanes
                for dd in range(_D):
                    plsc.addupdate_scatter(acc, [widx16 + dd * (_C * 16)],
                                           vd[u][dd])
                plsc.addupdate_scatter(cntacc, [widx16], ones)
            return carry

        lax.fori_loop(0, _N // (16 * _U), gbody, 0)

    pltpu.sync_copy(acc, psum_hbm.at[wid])
    pltpu.sync_copy(cntacc, pcnt_hbm.at[wid])


@functools.cache
def _sc_call():
    return functools.partial(
        pl.kernel,
        mesh=plsc.VectorSubcoreMesh(core_axis_name="c", subcore_axis_name="s"),
        compiler_params=pltpu.CompilerParams(needs_layout_passes=False),
        out_type=[
            jax.ShapeDtypeStruct((_NW, _D * _C * 16), jnp.float32),
            jax.ShapeDtypeStruct((_NW, _C * 16), jnp.float32),
        ],
        scratch_types=[
            pltpu.VMEM((_D, 1, _N), jnp.float32),
            pltpu.VMEM((_D * _C * 16,), jnp.float32),
            pltpu.VMEM((_C * 16,), jnp.float32),
            pltpu.VMEM((16 + _C * _D,), jnp.float32),
            pltpu.VMEM((_C * _D, 16), jnp.float32),
        ],
    )(_sc_body)


def kernel(inputs, W, digit_caps):
    x = inputs.reshape(_B, _N, _ID)
    xt = x.transpose(2, 0, 1)
    wt = W.transpose(1, 2, 0)
    ut, usum = _tc_call(xt, wt)
    psum, pcnt = _sc_call()(ut, digit_caps.reshape(-1))

    sum_u = jnp.sum(psum.reshape(_NW, _D, _C, 16), axis=(0, 3)).T
    cnt = jnp.sum(pcnt.reshape(_NW, _C, 16), axis=(0, 2))
    updates = (sum_u - cnt[:, None] * digit_caps) / (_B * _NM)
    new_dc = digit_caps + updates
    ubar = jnp.sum(usum.reshape(_M, _D, _B), axis=0).T / _NM
    output = ubar @ new_dc.T
    return output, new_dc


# speedup vs baseline: 2.4488x; 1.1296x over previous
import functools

import jax
import jax.numpy as jnp
from jax import lax
from jax.experimental import pallas as pl
from jax.experimental.pallas import tpu as pltpu
from jax.experimental.pallas import tpu_sc as plsc

_B, _N, _ID = 16, 8192, 8
_C, _D, _M = 10, 8, 4
_K = _D * _M
_NM = _N * _M
_JB = 512
_NW = 32


def _tc_body(xt_ref, wt_ref, dc_ref, ut_ref, win_ref, usum_ref):
    xb = xt_ref[...].astype(jnp.bfloat16).astype(jnp.float32)
    wb = wt_ref[...].astype(jnp.bfloat16).astype(jnp.float32)
    acc = wb[0][:, None, :] * xb[0][None, :, :]
    for i in range(1, _ID):
        acc = acc + wb[i][:, None, :] * xb[i][None, :, :]
    ut_ref[...] = acc

    @pl.when(pl.program_id(0) == 0)
    def _():
        usum_ref[...] = jnp.zeros_like(usum_ref)

    usum_ref[...] += jnp.sum(acc, axis=2)

    for m in range(_M):
        best = None
        widx = None
        for c in range(_C):
            s = acc[m * _D] * dc_ref[c, 0]
            for dd in range(1, _D):
                s = s + acc[m * _D + dd] * dc_ref[c, dd]
            if c == 0:
                best = s
                widx = jnp.zeros((_B, _JB), jnp.int32)
            else:
                gt = s > best
                best = jnp.where(gt, s, best)
                widx = jnp.where(gt, c, widx)
        win_ref[m] = widx


_tc_call = pl.pallas_call(
    _tc_body,
    grid=(_N // _JB,),
    in_specs=[
        pl.BlockSpec((_ID, _B, _JB), lambda j: (0, 0, j)),
        pl.BlockSpec((_ID, _K, _JB), lambda j: (0, 0, j)),
        pl.BlockSpec(memory_space=pltpu.SMEM),
    ],
    out_specs=[
        pl.BlockSpec((_K, _B, _JB), lambda j: (0, 0, j)),
        pl.BlockSpec((_M, _B, _JB), lambda j: (0, 0, j)),
        pl.BlockSpec((_K, _B), lambda j: (0, 0)),
    ],
    out_shape=[
        jax.ShapeDtypeStruct((_K, _B, _N), jnp.float32),
        jax.ShapeDtypeStruct((_M, _B, _N), jnp.int32),
        jax.ShapeDtypeStruct((_K, _B), jnp.float32),
    ],
)


_U = 4


def _sc_body(ut_hbm, win_hbm, psum_hbm, pcnt_hbm, buf, wbuf, acc, cntacc):
    wid = lax.axis_index("s") * 2 + lax.axis_index("c")
    lanes = jnp.arange(16, dtype=jnp.int32)
    ones = jnp.ones((16,), jnp.float32)

    for t in range(_D * _C):
        acc[pl.ds(t * 16, 16)] = jnp.zeros((16,), jnp.float32)
    for c in range(_C):
        cntacc[pl.ds(c * 16, 16)] = jnp.zeros((16,), jnp.float32)

    for cc in range(2):
        combo = wid * 2 + cc
        b = combo // _M
        m = combo % _M
        pltpu.sync_copy(ut_hbm.at[pl.ds(m * _D, _D), pl.ds(b, 1), :], buf)
        pltpu.sync_copy(win_hbm.at[pl.ds(m, 1), pl.ds(b, 1), :], wbuf)

        def gbody(g, carry):
            base = g * (16 * _U)
            for u in range(_U):
                sl = pl.ds(base + u * 16, 16)
                widx16 = wbuf[0, 0, sl] * 16 + lanes
                for dd in range(_D):
                    plsc.addupdate_scatter(acc, [widx16 + dd * (_C * 16)],
                                           buf[dd, 0, sl])
                plsc.addupdate_scatter(cntacc, [widx16], ones)
            return carry

        lax.fori_loop(0, _N // (16 * _U), gbody, 0)

    pltpu.sync_copy(acc, psum_hbm.at[wid])
    pltpu.sync_copy(cntacc, pcnt_hbm.at[wid])


@functools.cache
def _sc_call():
    return functools.partial(
        pl.kernel,
        mesh=plsc.VectorSubcoreMesh(core_axis_name="c", subcore_axis_name="s"),
        compiler_params=pltpu.CompilerParams(needs_layout_passes=False),
        out_type=[
            jax.ShapeDtypeStruct((_NW, _D * _C * 16), jnp.float32),
            jax.ShapeDtypeStruct((_NW, _C * 16), jnp.float32),
        ],
        scratch_types=[
            pltpu.VMEM((_D, 1, _N), jnp.float32),
            pltpu.VMEM((1, 1, _N), jnp.int32),
            pltpu.VMEM((_D * _C * 16,), jnp.float32),
            pltpu.VMEM((_C * 16,), jnp.float32),
        ],
    )(_sc_body)


def kernel(inputs, W, digit_caps):
    x = inputs.reshape(_B, _N, _ID)
    xt = x.transpose(2, 0, 1)
    wt = W.transpose(1, 2, 0)
    ut, win, usum = _tc_call(xt, wt, digit_caps)
    psum, pcnt = _sc_call()(ut, win)

    sum_u = jnp.sum(psum.reshape(_NW, _D, _C, 16), axis=(0, 3)).T
    cnt = jnp.sum(pcnt.reshape(_NW, _C, 16), axis=(0, 2))
    updates = (sum_u - cnt[:, None] * digit_caps) / (_B * _NM)
    new_dc = digit_caps + updates
    ubar = jnp.sum(usum.reshape(_M, _D, _B), axis=0).T / _NM
    output = ubar @ new_dc.T
    return output, new_dc


# speedup vs baseline: 2.5359x; 1.0356x over previous
import functools

import jax
import jax.numpy as jnp
from jax import lax
from jax.experimental import pallas as pl
from jax.experimental.pallas import tpu as pltpu
from jax.experimental.pallas import tpu_sc as plsc

_B, _N, _ID = 16, 8192, 8
_C, _D, _M = 10, 8, 4
_K = _D * _M
_NM = _N * _M
_JB = 512
_NW = 32


def _tc_body(xt_ref, wt_ref, dc_ref, ut_ref, win_ref, usum_ref):
    xb = xt_ref[...].astype(jnp.bfloat16).astype(jnp.float32)
    wb = wt_ref[...].astype(jnp.bfloat16).astype(jnp.float32)
    acc = wb[0][:, None, :] * xb[0][None, :, :]
    for i in range(1, _ID):
        acc = acc + wb[i][:, None, :] * xb[i][None, :, :]
    ut_ref[...] = acc

    @pl.when(pl.program_id(0) == 0)
    def _():
        usum_ref[...] = jnp.zeros_like(usum_ref)

    usum_ref[...] += jnp.sum(acc, axis=2)

    for m in range(_M):
        best = None
        widx = None
        for c in range(_C):
            s = acc[m * _D] * dc_ref[c, 0]
            for dd in range(1, _D):
                s = s + acc[m * _D + dd] * dc_ref[c, dd]
            if c == 0:
                best = s
                widx = jnp.zeros((_B, _JB), jnp.int32)
            else:
                gt = s > best
                best = jnp.where(gt, s, best)
                widx = jnp.where(gt, c, widx)
        win_ref[m] = widx


_tc_call = pl.pallas_call(
    _tc_body,
    grid=(_N // _JB,),
    in_specs=[
        pl.BlockSpec((_ID, _B, _JB), lambda j: (0, 0, j)),
        pl.BlockSpec((_ID, _K, _JB), lambda j: (0, 0, j)),
        pl.BlockSpec(memory_space=pltpu.SMEM),
    ],
    out_specs=[
        pl.BlockSpec((_K, _B, _JB), lambda j: (0, 0, j)),
        pl.BlockSpec((_M, _B, _JB), lambda j: (0, 0, j)),
        pl.BlockSpec((_K, _B), lambda j: (0, 0)),
    ],
    out_shape=[
        jax.ShapeDtypeStruct((_K, _B, _N), jnp.float32),
        jax.ShapeDtypeStruct((_M, _B, _N), jnp.int32),
        jax.ShapeDtypeStruct((_K, _B), jnp.float32),
    ],
)


_U = 8
_CH = _N // 2


def _sc_body(ut_hbm, win_hbm, psum_hbm, pcnt_hbm, dbuf, wbuf, acc, cntacc,
             sd0, sd1, sw0, sw1):
    wid = lax.axis_index("s") * 2 + lax.axis_index("c")
    lanes = jnp.arange(16, dtype=jnp.int32)
    ones = jnp.ones((16,), jnp.float32)
    sd = [sd0, sd1]
    sw = [sw0, sw1]

    for t in range(_D * _C):
        acc[pl.ds(t * 16, 16)] = jnp.zeros((16,), jnp.float32)
    for c in range(_C):
        cntacc[pl.ds(c * 16, 16)] = jnp.zeros((16,), jnp.float32)

    chunks = [(cc, h) for cc in range(2) for h in range(2)]
    hd = [None, None]
    hw = [None, None]

    def start(t):
        cc, h = chunks[t]
        combo = wid * 2 + cc
        b = combo // _M
        m = combo % _M
        s = t % 2
        hd[s] = pltpu.async_copy(
            ut_hbm.at[pl.ds(m * _D, _D), pl.ds(b, 1), pl.ds(h * _CH, _CH)],
            dbuf.at[s], sd[s])
        hw[s] = pltpu.async_copy(
            win_hbm.at[pl.ds(m, 1), pl.ds(b, 1), pl.ds(h * _CH, _CH)],
            wbuf.at[s], sw[s])

    start(0)
    for t in range(len(chunks)):
        s = t % 2
        if t + 1 < len(chunks):
            start(t + 1)
        hd[s].wait()
        hw[s].wait()

        def gbody(g, carry):
            base = g * (16 * _U)
            for u in range(_U):
                sl = pl.ds(base + u * 16, 16)
                widx16 = wbuf[s, 0, 0, sl] * 16 + lanes
                for dd in range(_D):
                    plsc.addupdate_scatter(acc, [widx16 + dd * (_C * 16)],
                                           dbuf[s, dd, 0, sl])
                plsc.addupdate_scatter(cntacc, [widx16], ones)
            return carry

        lax.fori_loop(0, _CH // (16 * _U), gbody, 0)

    pltpu.sync_copy(acc, psum_hbm.at[wid])
    pltpu.sync_copy(cntacc, pcnt_hbm.at[wid])


@functools.cache
def _sc_call():
    return functools.partial(
        pl.kernel,
        mesh=plsc.VectorSubcoreMesh(core_axis_name="c", subcore_axis_name="s"),
        compiler_params=pltpu.CompilerParams(needs_layout_passes=False),
        out_type=[
            jax.ShapeDtypeStruct((_NW, _D * _C * 16), jnp.float32),
            jax.ShapeDtypeStruct((_NW, _C * 16), jnp.float32),
        ],
        scratch_types=[
            pltpu.VMEM((2, _D, 1, _CH), jnp.float32),
            pltpu.VMEM((2, 1, 1, _CH), jnp.int32),
            pltpu.VMEM((_D * _C * 16,), jnp.float32),
            pltpu.VMEM((_C * 16,), jnp.float32),
            pltpu.SemaphoreType.DMA,
            pltpu.SemaphoreType.DMA,
            pltpu.SemaphoreType.DMA,
            pltpu.SemaphoreType.DMA,
        ],
    )(_sc_body)


def kernel(inputs, W, digit_caps):
    x = inputs.reshape(_B, _N, _ID)
    xt = x.transpose(2, 0, 1)
    wt = W.transpose(1, 2, 0)
    ut, win, usum = _tc_call(xt, wt, digit_caps)
    psum, pcnt = _sc_call()(ut, win)

    sum_u = jnp.sum(psum.reshape(_NW, _D, _C, 16), axis=(0, 3)).T
    cnt = jnp.sum(pcnt.reshape(_NW, _C, 16), axis=(0, 2))
    updates = (sum_u - cnt[:, None] * digit_caps) / (_B * _NM)
    new_dc = digit_caps + updates
    ubar = jnp.sum(usum.reshape(_M, _D, _B), axis=0).T / _NM
    output = ubar @ new_dc.T
    return output, new_dc


# speedup vs baseline: 2.5425x; 1.0026x over previous
import functools

import jax
import jax.numpy as jnp
from jax import lax
from jax.experimental import pallas as pl
from jax.experimental.pallas import tpu as pltpu
from jax.experimental.pallas import tpu_sc as plsc

_B, _N, _ID = 16, 8192, 8
_C, _D, _M = 10, 8, 4
_K = _D * _M
_NM = _N * _M
_JB = 512
_NW = 32
_JN = _N // 2


def _tc_body(xt_ref, wt_ref, dc_ref, ut_ref, win_ref, usum_ref):
    xb = xt_ref[...].astype(jnp.bfloat16).astype(jnp.float32)
    wb = wt_ref[...].astype(jnp.bfloat16).astype(jnp.float32)
    acc = wb[0][:, None, :] * xb[0][None, :, :]
    for i in range(1, _ID):
        acc = acc + wb[i][:, None, :] * xb[i][None, :, :]
    ut_ref[...] = acc

    @pl.when(pl.program_id(0) == 0)
    def _():
        usum_ref[...] = jnp.zeros_like(usum_ref)

    usum_ref[...] += jnp.sum(acc, axis=2)

    for m in range(_M):
        best = None
        widx = None
        for c in range(_C):
            s = acc[m * _D] * dc_ref[c, 0]
            for dd in range(1, _D):
                s = s + acc[m * _D + dd] * dc_ref[c, dd]
            if c == 0:
                best = s
                widx = jnp.zeros((_B, _JB), jnp.int32)
            else:
                gt = s > best
                best = jnp.where(gt, s, best)
                widx = jnp.where(gt, c, widx)
        win_ref[m] = widx


_tc_call = pl.pallas_call(
    _tc_body,
    grid=(_JN // _JB,),
    in_specs=[
        pl.BlockSpec((_ID, _B, _JB), lambda j: (0, 0, j)),
        pl.BlockSpec((_ID, _K, _JB), lambda j: (0, 0, j)),
        pl.BlockSpec(memory_space=pltpu.SMEM),
    ],
    out_specs=[
        pl.BlockSpec((_K, _B, _JB), lambda j: (0, 0, j)),
        pl.BlockSpec((_M, _B, _JB), lambda j: (0, 0, j)),
        pl.BlockSpec((_K, _B), lambda j: (0, 0)),
    ],
    out_shape=[
        jax.ShapeDtypeStruct((_K, _B, _JN), jnp.float32),
        jax.ShapeDtypeStruct((_M, _B, _JN), jnp.int32),
        jax.ShapeDtypeStruct((_K, _B), jnp.float32),
    ],
)


_U = 8
_CH = _JN


def _sc_body(ut_hbm, win_hbm, psum_hbm, pcnt_hbm, dbuf, wbuf, acc, cntacc,
             sd0, sd1, sw0, sw1):
    wid = lax.axis_index("s") * 2 + lax.axis_index("c")
    lanes = jnp.arange(16, dtype=jnp.int32)
    ones = jnp.ones((16,), jnp.float32)
    sd = [sd0, sd1]
    sw = [sw0, sw1]

    for t in range(_D * _C):
        acc[pl.ds(t * 16, 16)] = jnp.zeros((16,), jnp.float32)
    for c in range(_C):
        cntacc[pl.ds(c * 16, 16)] = jnp.zeros((16,), jnp.float32)

    chunks = [(cc, 0) for cc in range(2)]
    hd = [None, None]
    hw = [None, None]

    def start(t):
        cc, h = chunks[t]
        combo = wid * 2 + cc
        b = combo // _M
        m = combo % _M
        s = t % 2
        hd[s] = pltpu.async_copy(
            ut_hbm.at[pl.ds(m * _D, _D), pl.ds(b, 1), pl.ds(h * _CH, _CH)],
            dbuf.at[s], sd[s])
        hw[s] = pltpu.async_copy(
            win_hbm.at[pl.ds(m, 1), pl.ds(b, 1), pl.ds(h * _CH, _CH)],
            wbuf.at[s], sw[s])

    start(0)
    for t in range(len(chunks)):
        s = t % 2
        if t + 1 < len(chunks):
            start(t + 1)
        hd[s].wait()
        hw[s].wait()

        def gbody(g, carry):
            base = g * (16 * _U)
            for u in range(_U):
                sl = pl.ds(base + u * 16, 16)
                widx16 = wbuf[s, 0, 0, sl] * 16 + lanes
                for dd in range(_D):
                    plsc.addupdate_scatter(acc, [widx16 + dd * (_C * 16)],
                                           dbuf[s, dd, 0, sl])
                plsc.addupdate_scatter(cntacc, [widx16], ones)
            return carry

        lax.fori_loop(0, _CH // (16 * _U), gbody, 0)

    pltpu.sync_copy(acc, psum_hbm.at[wid])
    pltpu.sync_copy(cntacc, pcnt_hbm.at[wid])


@functools.cache
def _sc_call():
    return functools.partial(
        pl.kernel,
        mesh=plsc.VectorSubcoreMesh(core_axis_name="c", subcore_axis_name="s"),
        compiler_params=pltpu.CompilerParams(needs_layout_passes=False),
        out_type=[
            jax.ShapeDtypeStruct((_NW, _D * _C * 16), jnp.float32),
            jax.ShapeDtypeStruct((_NW, _C * 16), jnp.float32),
        ],
        scratch_types=[
            pltpu.VMEM((2, _D, 1, _CH), jnp.float32),
            pltpu.VMEM((2, 1, 1, _CH), jnp.int32),
            pltpu.VMEM((_D * _C * 16,), jnp.float32),
            pltpu.VMEM((_C * 16,), jnp.float32),
            pltpu.SemaphoreType.DMA,
            pltpu.SemaphoreType.DMA,
            pltpu.SemaphoreType.DMA,
            pltpu.SemaphoreType.DMA,
        ],
    )(_sc_body)


def kernel(inputs, W, digit_caps):
    x = inputs.reshape(_B, _N, _ID)
    xt = x.transpose(2, 0, 1)
    wt = W.transpose(1, 2, 0)
    sc = _sc_call()
    ut0, win0, usum0 = _tc_call(xt[:, :, :_JN], wt[:, :, :_JN], digit_caps)
    ut1, win1, usum1 = _tc_call(xt[:, :, _JN:], wt[:, :, _JN:], digit_caps)
    psum0, pcnt0 = sc(ut0, win0)
    psum1, pcnt1 = sc(ut1, win1)
    psum = psum0 + psum1
    pcnt = pcnt0 + pcnt1
    usum = usum0 + usum1

    sum_u = jnp.sum(psum.reshape(_NW, _D, _C, 16), axis=(0, 3)).T
    cnt = jnp.sum(pcnt.reshape(_NW, _C, 16), axis=(0, 2))
    updates = (sum_u - cnt[:, None] * digit_caps) / (_B * _NM)
    new_dc = digit_caps + updates
    ubar = jnp.sum(usum.reshape(_M, _D, _B), axis=0).T / _NM
    output = ubar @ new_dc.T
    return output, new_dc


# speedup vs baseline: 2.9132x; 1.1458x over previous
import functools

import jax
import jax.numpy as jnp
from jax import lax
from jax.experimental import pallas as pl
from jax.experimental.pallas import tpu as pltpu
from jax.experimental.pallas import tpu_sc as plsc

_B, _N, _ID = 16, 8192, 8
_C, _D, _M = 10, 8, 4
_K = _D * _M
_NM = _N * _M
_JB = 512
_NW = 32
_JN = _N // 2


def _tc_body(xt_ref, wt_ref, dc_ref, ut_ref, win_ref, usum_ref):
    xb = xt_ref[...].astype(jnp.bfloat16).astype(jnp.float32)
    wb = wt_ref[...].astype(jnp.bfloat16).astype(jnp.float32)
    acc = wb[0][:, None, :] * xb[0][None, :, :]
    for i in range(1, _ID):
        acc = acc + wb[i][:, None, :] * xb[i][None, :, :]
    ut_ref[...] = acc

    @pl.when(pl.program_id(0) == 0)
    def _():
        usum_ref[...] = jnp.zeros_like(usum_ref)

    usum_ref[...] += jnp.sum(acc, axis=2)

    for m in range(_M):
        best = None
        widx = None
        for c in range(_C):
            s = acc[m * _D] * dc_ref[c, 0]
            for dd in range(1, _D):
                s = s + acc[m * _D + dd] * dc_ref[c, dd]
            if c == 0:
                best = s
                widx = jnp.zeros((_B, _JB), jnp.int32)
            else:
                gt = s > best
                best = jnp.where(gt, s, best)
                widx = jnp.where(gt, c, widx)
        win_ref[m] = widx


_tc_call = pl.pallas_call(
    _tc_body,
    grid=(_JN // _JB,),
    in_specs=[
        pl.BlockSpec((_ID, _B, _JB), lambda j: (0, 0, j)),
        pl.BlockSpec((_ID, _K, _JB), lambda j: (0, 0, j)),
        pl.BlockSpec(memory_space=pltpu.SMEM),
    ],
    out_specs=[
        pl.BlockSpec((_K, _B, _JB), lambda j: (0, 0, j)),
        pl.BlockSpec((_M, _B, _JB), lambda j: (0, 0, j)),
        pl.BlockSpec((_K, _B), lambda j: (0, 0)),
    ],
    out_shape=[
        jax.ShapeDtypeStruct((_K, _B, _JN), jnp.float32),
        jax.ShapeDtypeStruct((_M, _B, _JN), jnp.int32),
        jax.ShapeDtypeStruct((_K, _B), jnp.float32),
    ],
)


_U = 8
_CH = _JN


def _sc_body(ut_hbm, win_hbm, psum_hbm, pcnt_hbm, dbuf, wbuf, acc, cntacc,
             sd0, sd1, sw0, sw1):
    wid = lax.axis_index("s") * 2 + lax.axis_index("c")
    lanes = jnp.arange(16, dtype=jnp.int32)
    ones = jnp.ones((16,), jnp.float32)
    sd = [sd0, sd1]
    sw = [sw0, sw1]

    for t in range(_D * _C):
        acc[pl.ds(t * 16, 16)] = jnp.zeros((16,), jnp.float32)
    for c in range(_C):
        cntacc[pl.ds(c * 16, 16)] = jnp.zeros((16,), jnp.float32)

    chunks = [(cc, 0) for cc in range(2)]
    hd = [None, None]
    hw = [None, None]

    def start(t):
        cc, h = chunks[t]
        combo = wid * 2 + cc
        b = combo // _M
        m = combo % _M
        s = t % 2
        hd[s] = pltpu.async_copy(
            ut_hbm.at[pl.ds(m * _D, _D), pl.ds(b, 1), pl.ds(h * _CH, _CH)],
            dbuf.at[s], sd[s])
        hw[s] = pltpu.async_copy(
            win_hbm.at[pl.ds(m, 1), pl.ds(b, 1), pl.ds(h * _CH, _CH)],
            wbuf.at[s], sw[s])

    start(0)
    for t in range(len(chunks)):
        s = t % 2
        if t + 1 < len(chunks):
            start(t + 1)
        hd[s].wait()
        hw[s].wait()

        @plsc.parallel_loop(0, _CH // (16 * _U), unroll=2)
        def gbody(g):
            base = g * (16 * _U)
            for u in range(_U):
                sl = pl.ds(base + u * 16, 16)
                widx16 = wbuf[s, 0, 0, sl] * 16 + lanes
                for dd in range(_D):
                    plsc.addupdate_scatter(acc, [widx16 + dd * (_C * 16)],
                                           dbuf[s, dd, 0, sl])
                plsc.addupdate_scatter(cntacc, [widx16], ones)

    pltpu.sync_copy(acc, psum_hbm.at[wid])
    pltpu.sync_copy(cntacc, pcnt_hbm.at[wid])


@functools.cache
def _sc_call():
    return functools.partial(
        pl.kernel,
        mesh=plsc.VectorSubcoreMesh(core_axis_name="c", subcore_axis_name="s"),
        compiler_params=pltpu.CompilerParams(needs_layout_passes=False),
        out_type=[
            jax.ShapeDtypeStruct((_NW, _D * _C * 16), jnp.float32),
            jax.ShapeDtypeStruct((_NW, _C * 16), jnp.float32),
        ],
        scratch_types=[
            pltpu.VMEM((2, _D, 1, _CH), jnp.float32),
            pltpu.VMEM((2, 1, 1, _CH), jnp.int32),
            pltpu.VMEM((_D * _C * 16,), jnp.float32),
            pltpu.VMEM((_C * 16,), jnp.float32),
            pltpu.SemaphoreType.DMA,
            pltpu.SemaphoreType.DMA,
            pltpu.SemaphoreType.DMA,
            pltpu.SemaphoreType.DMA,
        ],
    )(_sc_body)


def kernel(inputs, W, digit_caps):
    x = inputs.reshape(_B, _N, _ID)
    xt = x.transpose(2, 0, 1)
    wt = W.transpose(1, 2, 0)
    sc = _sc_call()
    ut0, win0, usum0 = _tc_call(xt[:, :, :_JN], wt[:, :, :_JN], digit_caps)
    ut1, win1, usum1 = _tc_call(xt[:, :, _JN:], wt[:, :, _JN:], digit_caps)
    psum0, pcnt0 = sc(ut0, win0)
    psum1, pcnt1 = sc(ut1, win1)
    psum = psum0 + psum1
    pcnt = pcnt0 + pcnt1
    usum = usum0 + usum1

    sum_u = jnp.sum(psum.reshape(_NW, _D, _C, 16), axis=(0, 3)).T
    cnt = jnp.sum(pcnt.reshape(_NW, _C, 16), axis=(0, 2))
    updates = (sum_u - cnt[:, None] * digit_caps) / (_B * _NM)
    new_dc = digit_caps + updates
    ubar = jnp.sum(usum.reshape(_M, _D, _B), axis=0).T / _NM
    output = ubar @ new_dc.T
    return output, new_dc
